# baseline (speedup 1.0000x reference)
import functools

import jax
import jax.numpy as jnp
from jax import lax
from jax.experimental import pallas as pl
from jax.experimental.pallas import tpu as pltpu
from jax.experimental.pallas import tpu_sc as plsc

N_NODES = 10000
N_EDGES = 320000
D_NODE = 128
D_EDGE = 16
D_HIDDEN = 128
D_OUT = 128
N_GRAPHS = 64

NC = 2
NS = 16
NW = NC * NS
CH = 128
NCHUNK = N_EDGES // CH
MAIN = 2 * ((NCHUNK // NW) // 2)
HALF = MAIN // 2
EXTRA = NCHUNK - MAIN * NW
NPAD = 10112
RPW = NPAD // NS

_MESH = plsc.VectorSubcoreMesh(core_axis_name="c", subcore_axis_name="s",
                               num_cores=NC, num_subcores=NS)


def _chunk_off(q, w):
    return pl.multiple_of(q * (NW * CH) + w * CH, CH)


def _sc_x_body(src_hbm, dst_hbm, x_hbm, zx_hbm, aggx_out,
               src_v, dst_v, rows_v, aggx_sh, sg0, sg1):
    c = lax.axis_index("c")
    s = lax.axis_index("s")
    w = s * NC + c

    pltpu.sync_copy(zx_hbm, aggx_sh.at[pl.ds(s * RPW, RPW)])
    plsc.subcore_barrier()

    sg = (sg0, sg1)

    def load(q, b):
        off = _chunk_off(q, w)
        pltpu.sync_copy(src_hbm.at[pl.ds(off, CH)], src_v.at[b])
        pltpu.sync_copy(dst_hbm.at[pl.ds(off, CH)], dst_v.at[b])
        pltpu.async_copy(x_hbm.at[src_v.at[b]], rows_v.at[b], sg[b])

    def consume(b):
        pltpu.make_async_copy(x_hbm.at[src_v.at[b]], rows_v.at[b],
                              sg[b]).wait()
        pltpu.sync_copy(rows_v.at[b], aggx_sh.at[dst_v.at[b]], add=True)

    load(0, 0)

    def outer(i, carry):
        load(2 * i + 1, 1)
        consume(0)

        @pl.when(i < HALF - 1)
        def _():
            load(2 * i + 2, 0)

        consume(1)
        return carry

    lax.fori_loop(0, HALF, outer, 0)

    @pl.when(w < EXTRA)
    def _():
        load(MAIN + 0, 0)
        consume(0)

    plsc.subcore_barrier()
    pltpu.sync_copy(aggx_sh.at[pl.ds(s * RPW, RPW)],
                    aggx_out.at[c, pl.ds(s * RPW, RPW)])


_sc_scatter_x = functools.partial(
    pl.kernel,
    out_type=jax.ShapeDtypeStruct((NC, NPAD, D_NODE), jnp.float32),
    mesh=_MESH,
    scratch_types=[
        pltpu.VMEM((2, CH), jnp.int32),
        pltpu.VMEM((2, CH), jnp.int32),
        pltpu.VMEM((2, CH, D_NODE), jnp.float32),
        pltpu.VMEM_SHARED((NPAD, D_NODE), jnp.float32),
        pltpu.SemaphoreType.DMA,
        pltpu.SemaphoreType.DMA,
    ],
    compiler_params=pltpu.CompilerParams(use_tc_tiling_on_sc=False),
)(_sc_x_body)


def _sc_ea_body(dst_hbm, ea_hbm, ze_hbm, agge_out,
                dst_v, ea_v, agge_sh, se0, se1):
    c = lax.axis_index("c")
    s = lax.axis_index("s")
    w = s * NC + c

    pltpu.sync_copy(ze_hbm, agge_sh.at[pl.ds(s * RPW, RPW)])
    plsc.subcore_barrier()

    se = (se0, se1)

    def load(q, b):
        off = _chunk_off(q, w)
        pltpu.sync_copy(dst_hbm.at[pl.ds(off, CH)], dst_v.at[b])
        pltpu.async_copy(ea_hbm.at[pl.ds(off, CH)], ea_v.at[b], se[b])

    def consume(b):
        pltpu.make_async_copy(ea_hbm.at[pl.ds(0, CH)], ea_v.at[b],
                              se[b]).wait()
        pltpu.sync_copy(ea_v.at[b], agge_sh.at[dst_v.at[b]], add=True)

    load(0, 0)

    def outer(i, carry):
        load(2 * i + 1, 1)
        consume(0)

        @pl.when(i < HALF - 1)
        def _():
            load(2 * i + 2, 0)

        consume(1)
        return carry

    lax.fori_loop(0, HALF, outer, 0)

    @pl.when(w < EXTRA)
    def _():
        load(MAIN, 0)
        consume(0)

    plsc.subcore_barrier()
    pltpu.sync_copy(agge_sh.at[pl.ds(s * RPW, RPW)],
                    agge_out.at[c, pl.ds(s * RPW, RPW)])


_sc_scatter_ea = functools.partial(
    pl.kernel,
    out_type=jax.ShapeDtypeStruct((NC, NPAD, D_EDGE), jnp.float32),
    mesh=_MESH,
    scratch_types=[
        pltpu.VMEM((2, CH), jnp.int32),
        pltpu.VMEM((2, CH, D_EDGE), jnp.float32),
        pltpu.VMEM_SHARED((NPAD, D_EDGE), jnp.float32),
        pltpu.SemaphoreType.DMA,
        pltpu.SemaphoreType.DMA,
    ],
    compiler_params=pltpu.CompilerParams(use_tc_tiling_on_sc=False),
)(_sc_ea_body)


TC_BLOCK = 1264
TC_STEPS = NPAD // TC_BLOCK


def _tc_body(aggx_ref, agge_ref, batch_ref, w1a_ref, w1b_ref,
             b1_ref, w2_ref, b2_ref, out_ref, sum_sc, cnt_sc):
    i = pl.program_id(0)
    ax = aggx_ref[0] + aggx_ref[1]
    ae = agge_ref[0] + agge_ref[1]
    h = jnp.dot(ax, w1a_ref[...], preferred_element_type=jnp.float32)
    h += jnp.dot(ae, w1b_ref[...], preferred_element_type=jnp.float32)
    h = jnp.maximum(h + b1_ref[...], 0.0)
    h2 = jnp.dot(h, w2_ref[...], preferred_element_type=jnp.float32) + b2_ref[...]
    bt = batch_ref[0, 0, :]
    oh = (bt[:, None] == lax.broadcasted_iota(jnp.int32, (1, N_GRAPHS), 1))
    oh = oh.astype(jnp.float32)
    psum = lax.dot_general(oh, h2, (((0,), (0,)), ((), ())),
                           preferred_element_type=jnp.float32)
    ones = jnp.ones((TC_BLOCK, 1), jnp.float32)
    pcnt = lax.dot_general(oh, ones, (((0,), (0,)), ((), ())),
                           preferred_element_type=jnp.float32)

    @pl.when(i == 0)
    def _():
        sum_sc[...] = psum
        cnt_sc[...] = pcnt

    @pl.when(i > 0)
    def _():
        sum_sc[...] += psum
        cnt_sc[...] += pcnt

    @pl.when(i == TC_STEPS - 1)
    def _():
        out_ref[...] = sum_sc[...] / jnp.maximum(cnt_sc[...], 1.0)


_tc_mlp_pool = pl.pallas_call(
    _tc_body,
    grid=(TC_STEPS,),
    in_specs=[
        pl.BlockSpec((NC, TC_BLOCK, D_NODE), lambda i: (0, i, 0)),
        pl.BlockSpec((NC, TC_BLOCK, D_EDGE), lambda i: (0, i, 0)),
        pl.BlockSpec((1, 1, TC_BLOCK), lambda i: (i, 0, 0)),
        pl.BlockSpec((D_NODE, D_HIDDEN), lambda i: (0, 0)),
        pl.BlockSpec((D_EDGE, D_HIDDEN), lambda i: (0, 0)),
        pl.BlockSpec((1, D_HIDDEN), lambda i: (0, 0)),
        pl.BlockSpec((D_HIDDEN, D_OUT), lambda i: (0, 0)),
        pl.BlockSpec((1, D_OUT), lambda i: (0, 0)),
    ],
    out_specs=pl.BlockSpec((N_GRAPHS, D_OUT), lambda i: (0, 0)),
    out_shape=jax.ShapeDtypeStruct((N_GRAPHS, D_OUT), jnp.float32),
    scratch_shapes=[
        pltpu.VMEM((N_GRAPHS, D_OUT), jnp.float32),
        pltpu.VMEM((N_GRAPHS, 1), jnp.float32),
    ],
    compiler_params=pltpu.CompilerParams(
        dimension_semantics=("arbitrary",)),
)


def kernel(x, edge_index, edge_attr, batch, W1, b1, W2, b2):
    src = edge_index[0].astype(jnp.int32)
    dst = edge_index[1].astype(jnp.int32)
    zx = jnp.zeros((RPW, D_NODE), jnp.float32)
    ze = jnp.zeros((RPW, D_EDGE), jnp.float32)

    aggx = _sc_scatter_x(src, dst, x, zx)
    agge = _sc_scatter_ea(dst, edge_attr, ze)

    batchp = jnp.full((NPAD,), N_GRAPHS, jnp.int32).at[:N_NODES].set(
        batch.astype(jnp.int32)).reshape(TC_STEPS, 1, TC_BLOCK)
    pooled = _tc_mlp_pool(aggx, agge, batchp, W1[:D_NODE], W1[D_NODE:],
                          b1.reshape(1, D_HIDDEN), W2, b2.reshape(1, D_OUT))
    return pooled


# speedup vs baseline: 1.3133x; 1.3133x over previous
import functools

import jax
import jax.numpy as jnp
from jax import lax
from jax.experimental import pallas as pl
from jax.experimental.pallas import tpu as pltpu
from jax.experimental.pallas import tpu_sc as plsc

N_NODES = 10000
N_EDGES = 320000
D_NODE = 128
D_EDGE = 16
D_HIDDEN = 128
D_OUT = 128
N_GRAPHS = 64

NC = 2
NS = 16
NW = NC * NS
CH = 128
NCHUNK = N_EDGES // CH
MAIN = 2 * ((NCHUNK // NW) // 2)
HALF = MAIN // 2
EXTRA = NCHUNK - MAIN * NW
NPAD = 10112
RPW = NPAD // NS

_MESH = plsc.VectorSubcoreMesh(core_axis_name="c", subcore_axis_name="s",
                               num_cores=NC, num_subcores=NS)


def _chunk_off(q, w):
    return pl.multiple_of(q * (NW * CH) + w * CH, CH)


def _sc_x_body(src_hbm, dst_hbm, x_hbm, zx_hbm, aggx_out,
               src_v, dst_v, rows_v, aggx_sh, sg0, sg1):
    c = lax.axis_index("c")
    s = lax.axis_index("s")
    w = s * NC + c

    pltpu.sync_copy(zx_hbm, aggx_sh.at[pl.ds(s * RPW, RPW)])
    plsc.subcore_barrier()

    sg = (sg0, sg1)

    def load(q, b):
        off = _chunk_off(q, w)
        pltpu.sync_copy(src_hbm.at[pl.ds(off, CH)], src_v.at[b])
        pltpu.sync_copy(dst_hbm.at[pl.ds(off, CH)], dst_v.at[b])
        pltpu.async_copy(x_hbm.at[src_v.at[b]], rows_v.at[b], sg[b])

    def consume(b):
        pltpu.make_async_copy(x_hbm.at[src_v.at[b]], rows_v.at[b],
                              sg[b]).wait()
        pltpu.sync_copy(rows_v.at[b], aggx_sh.at[dst_v.at[b]], add=True)

    load(0, 0)

    def outer(i, carry):
        load(2 * i + 1, 1)
        consume(0)

        @pl.when(i < HALF - 1)
        def _():
            load(2 * i + 2, 0)

        consume(1)
        return carry

    lax.fori_loop(0, HALF, outer, 0)

    @pl.when(w < EXTRA)
    def _():
        load(MAIN + 0, 0)
        consume(0)

    plsc.subcore_barrier()
    pltpu.sync_copy(aggx_sh.at[pl.ds(s * RPW, RPW)],
                    aggx_out.at[c, pl.ds(s * RPW, RPW)])


_sc_scatter_x = functools.partial(
    pl.kernel,
    out_type=jax.ShapeDtypeStruct((NC, NPAD, D_NODE), jnp.float32),
    mesh=_MESH,
    scratch_types=[
        pltpu.VMEM((2, CH), jnp.int32),
        pltpu.VMEM((2, CH), jnp.int32),
        pltpu.VMEM((2, CH, D_NODE), jnp.float32),
        pltpu.VMEM_SHARED((NPAD, D_NODE), jnp.float32),
        pltpu.SemaphoreType.DMA,
        pltpu.SemaphoreType.DMA,
    ],
    compiler_params=pltpu.CompilerParams(use_tc_tiling_on_sc=False),
)(_sc_x_body)


def _sc_ea_body(dst_hbm, ea_hbm, ze_hbm, aggx_hbm, agge_out,
                dst_v, ea_v, agge_sh, dep_v, se0, se1):
    c = lax.axis_index("c")
    s = lax.axis_index("s")
    w = s * NC + c

    pltpu.sync_copy(aggx_hbm.at[0, pl.ds(0, 8)], dep_v)
    pltpu.sync_copy(ze_hbm, agge_sh.at[pl.ds(s * RPW, RPW)])
    plsc.subcore_barrier()

    se = (se0, se1)

    def load(q, b):
        off = _chunk_off(q, w)
        pltpu.sync_copy(dst_hbm.at[pl.ds(off, CH)], dst_v.at[b])
        pltpu.async_copy(ea_hbm.at[pl.ds(off, CH)], ea_v.at[b], se[b])

    def consume(b):
        pltpu.make_async_copy(ea_hbm.at[pl.ds(0, CH)], ea_v.at[b],
                              se[b]).wait()
        pltpu.sync_copy(ea_v.at[b], agge_sh.at[dst_v.at[b]], add=True)

    load(0, 0)

    def outer(i, carry):
        load(2 * i + 1, 1)
        consume(0)

        @pl.when(i < HALF - 1)
        def _():
            load(2 * i + 2, 0)

        consume(1)
        return carry

    lax.fori_loop(0, HALF, outer, 0)

    @pl.when(w < EXTRA)
    def _():
        load(MAIN, 0)
        consume(0)

    plsc.subcore_barrier()
    pltpu.sync_copy(agge_sh.at[pl.ds(s * RPW, RPW)],
                    agge_out.at[c, pl.ds(s * RPW, RPW)])


_sc_scatter_ea = functools.partial(
    pl.kernel,
    out_type=jax.ShapeDtypeStruct((NC, NPAD, D_EDGE), jnp.float32),
    mesh=_MESH,
    scratch_types=[
        pltpu.VMEM((2, CH), jnp.int32),
        pltpu.VMEM((2, CH, D_EDGE), jnp.float32),
        pltpu.VMEM_SHARED((NPAD, D_EDGE), jnp.float32),
        pltpu.VMEM((8, D_NODE), jnp.float32),
        pltpu.SemaphoreType.DMA,
        pltpu.SemaphoreType.DMA,
    ],
    compiler_params=pltpu.CompilerParams(use_tc_tiling_on_sc=False),
)(_sc_ea_body)


TC_BLOCK = 1264
TC_STEPS = NPAD // TC_BLOCK


def _tc_body(aggx_ref, agge_ref, batch_ref, w1a_ref, w1b_ref,
             b1_ref, w2_ref, b2_ref, out_ref, sum_sc, cnt_sc):
    i = pl.program_id(0)
    ax = aggx_ref[0] + aggx_ref[1]
    ae = agge_ref[0] + agge_ref[1]
    h = jnp.dot(ax, w1a_ref[...], preferred_element_type=jnp.float32)
    h += jnp.dot(ae, w1b_ref[...], preferred_element_type=jnp.float32)
    h = jnp.maximum(h + b1_ref[...], 0.0)
    h2 = jnp.dot(h, w2_ref[...], preferred_element_type=jnp.float32) + b2_ref[...]
    bt = batch_ref[0, 0, :]
    oh = (bt[:, None] == lax.broadcasted_iota(jnp.int32, (1, N_GRAPHS), 1))
    oh = oh.astype(jnp.float32)
    psum = lax.dot_general(oh, h2, (((0,), (0,)), ((), ())),
                           preferred_element_type=jnp.float32)
    ones = jnp.ones((TC_BLOCK, 1), jnp.float32)
    pcnt = lax.dot_general(oh, ones, (((0,), (0,)), ((), ())),
                           preferred_element_type=jnp.float32)

    @pl.when(i == 0)
    def _():
        sum_sc[...] = psum
        cnt_sc[...] = pcnt

    @pl.when(i > 0)
    def _():
        sum_sc[...] += psum
        cnt_sc[...] += pcnt

    @pl.when(i == TC_STEPS - 1)
    def _():
        out_ref[...] = sum_sc[...] / jnp.maximum(cnt_sc[...], 1.0)


_tc_mlp_pool = pl.pallas_call(
    _tc_body,
    grid=(TC_STEPS,),
    in_specs=[
        pl.BlockSpec((NC, TC_BLOCK, D_NODE), lambda i: (0, i, 0)),
        pl.BlockSpec((NC, TC_BLOCK, D_EDGE), lambda i: (0, i, 0)),
        pl.BlockSpec((1, 1, TC_BLOCK), lambda i: (i, 0, 0)),
        pl.BlockSpec((D_NODE, D_HIDDEN), lambda i: (0, 0)),
        pl.BlockSpec((D_EDGE, D_HIDDEN), lambda i: (0, 0)),
        pl.BlockSpec((1, D_HIDDEN), lambda i: (0, 0)),
        pl.BlockSpec((D_HIDDEN, D_OUT), lambda i: (0, 0)),
        pl.BlockSpec((1, D_OUT), lambda i: (0, 0)),
    ],
    out_specs=pl.BlockSpec((N_GRAPHS, D_OUT), lambda i: (0, 0)),
    out_shape=jax.ShapeDtypeStruct((N_GRAPHS, D_OUT), jnp.float32),
    scratch_shapes=[
        pltpu.VMEM((N_GRAPHS, D_OUT), jnp.float32),
        pltpu.VMEM((N_GRAPHS, 1), jnp.float32),
    ],
    compiler_params=pltpu.CompilerParams(
        dimension_semantics=("arbitrary",)),
)


def kernel(x, edge_index, edge_attr, batch, W1, b1, W2, b2):
    src = edge_index[0].astype(jnp.int32)
    dst = edge_index[1].astype(jnp.int32)
    zx = jnp.zeros((RPW, D_NODE), jnp.float32)
    ze = jnp.zeros((RPW, D_EDGE), jnp.float32)

    aggx = _sc_scatter_x(src, dst, x, zx)
    agge = _sc_scatter_ea(dst, edge_attr, ze, aggx)

    batchp = jnp.full((NPAD,), N_GRAPHS, jnp.int32).at[:N_NODES].set(
        batch.astype(jnp.int32)).reshape(TC_STEPS, 1, TC_BLOCK)
    pooled = _tc_mlp_pool(aggx, agge, batchp, W1[:D_NODE], W1[D_NODE:],
                          b1.reshape(1, D_HIDDEN), W2, b2.reshape(1, D_OUT))
    return pooled
